# baseline (speedup 1.0000x reference)
import functools

import jax
import jax.numpy as jnp
from jax import lax
from jax.experimental import pallas as pl
from jax.experimental.pallas import tpu as pltpu
from jax.experimental.pallas import tpu_sc as plsc

K = 100000
D = 16
B = 16384

_NW = 32
_BPW = B // _NW
_SUB = 8
_G1 = B // _SUB
_L = 16
_CHUNK = 64
_NCHUNK = _BPW // _CHUNK

_TC_ROWS = 2048
_TC_G = _TC_ROWS // _SUB


def _make_gather_kernel():
  mesh = plsc.VectorSubcoreMesh(core_axis_name="c", subcore_axis_name="s")

  @functools.partial(
      pl.kernel,
      mesh=mesh,
      compiler_params=pltpu.CompilerParams(use_tc_tiling_on_sc=True),
      out_type=jax.ShapeDtypeStruct((_G1, _SUB, 128), jnp.float32),
      scratch_types=[
          pltpu.VMEM((_BPW,), jnp.int32),
          pltpu.VMEM((_CHUNK * _SUB, D), jnp.float32),
          pltpu.VMEM((_CHUNK // _SUB, _SUB, 128), jnp.float32),
          pltpu.SemaphoreType.DMA,
      ],
  )
  def k(a_hbm, m_hbm, g_hbm, idx_v, fetch_v, pack_v, sem):
    wid = lax.axis_index("s") * 2 + lax.axis_index("c")
    base = wid * _BPW
    pltpu.sync_copy(a_hbm.at[pl.ds(base, _BPW)], idx_v)

    def chunk_body(ch, _):
      def fire_body(q, _):
        a16 = idx_v[pl.ds(ch * _CHUNK + q * _L, _L)]
        t8 = jnp.bitwise_and(a16, jnp.int32(~7))
        copies = []
        for j in range(_L):
          src = m_hbm.at[pl.ds(pl.multiple_of(t8[j], _SUB), _SUB), :]
          dst = fetch_v.at[pl.ds((q * _L + j) * _SUB, _SUB), :]
          copies.append(pltpu.async_copy(src, dst, sem))
        for cp in copies:
          cp.wait()
        return _

      lax.fori_loop(0, _CHUNK // _L, fire_body, 0)

      def sel_body(r, _):
        s16 = jnp.bitwise_and(idx_v[pl.ds(ch * _CHUNK + r * _L, _L)], 7)
        for j in range(_L):
          i = r * _L + j
          pack_v[2 * r + j // _SUB, j % _SUB, pl.ds(0, D)] = (
              fetch_v[i * _SUB + s16[j], :])
        return _

      lax.fori_loop(0, _CHUNK // _L, sel_body, 0)
      pltpu.sync_copy(
          pack_v,
          g_hbm.at[pl.ds(wid * (_BPW // _SUB) + ch * (_CHUNK // _SUB),
                         _CHUNK // _SUB)])
      return _

    lax.fori_loop(0, _NCHUNK, chunk_body, 0)

  return k


_gather = _make_gather_kernel()


def _dot_sigmoid_body(x_ref, g_ref, o_ref):
  g = g_ref[...][:, :, :D].reshape(_TC_ROWS, D)
  p = x_ref[...] * g
  z = jnp.sum(p, axis=1)
  o_ref[...] = 1.0 / (1.0 + jnp.exp(-z))


_dot_sigmoid = pl.pallas_call(
    _dot_sigmoid_body,
    grid=(B // _TC_ROWS,),
    in_specs=[
        pl.BlockSpec((_TC_ROWS, D), lambda i: (i, 0)),
        pl.BlockSpec((_TC_G, _SUB, 128), lambda i: (i, 0, 0)),
    ],
    out_specs=pl.BlockSpec((_TC_ROWS,), lambda i: (i,)),
    out_shape=jax.ShapeDtypeStruct((B,), jnp.float32),
)


@jax.jit
def kernel(X, A, m):
  g3 = _gather(A.astype(jnp.int32), m)
  return _dot_sigmoid(X, g3)


# speedup vs baseline: 1.1429x; 1.1429x over previous
import functools

import jax
import jax.numpy as jnp
from jax import lax
from jax.experimental import pallas as pl
from jax.experimental.pallas import tpu as pltpu
from jax.experimental.pallas import tpu_sc as plsc

K = 100000
D = 16
B = 16384

_NW = 32
_BPW = B // _NW
_SUB = 8
_KT = K // _SUB
_G1 = B // _SUB
_L = 16
_NCHUNK = 2
_CHUNK = _BPW // _NCHUNK

_KPAD = 102400
_MCOLS = 2048
_MROWS = _MCOLS // _SUB
_KTP = _KPAD // _SUB

_TC_ROWS = 2048
_TC_G = _TC_ROWS // _SUB


def _repack_m_body(mt_ref, eye_ref, o_ref):
  t = jax.lax.dot_general(mt_ref[...], eye_ref[...], (((0,), (0,)), ((), ())),
                          preferred_element_type=jnp.float32)
  t8 = t.reshape(_MROWS, _SUB, D)
  parts = [t8[:, s, :] for s in range(_SUB)]
  o_ref[...] = jnp.concatenate(parts, axis=1)


_repack_m = pl.pallas_call(
    _repack_m_body,
    grid=(_KPAD // _MCOLS,),
    in_specs=[
        pl.BlockSpec((D, _MCOLS), lambda i: (0, i)),
        pl.BlockSpec((D, D), lambda i: (0, 0)),
    ],
    out_specs=pl.BlockSpec((_MROWS, 128), lambda i: (i, 0)),
    out_shape=jax.ShapeDtypeStruct((_KTP, 128), jnp.float32),
)


def _make_gather_kernel():
  mesh = plsc.VectorSubcoreMesh(core_axis_name="c", subcore_axis_name="s")

  @functools.partial(
      pl.kernel,
      mesh=mesh,
      compiler_params=pltpu.CompilerParams(use_tc_tiling_on_sc=False),
      out_type=jax.ShapeDtypeStruct((_G1, _SUB, 128), jnp.float32),
      scratch_types=[
          pltpu.VMEM((_BPW,), jnp.int32),
          pltpu.VMEM((_BPW,), jnp.int32),
          pltpu.VMEM((_CHUNK, 128), jnp.float32),
          pltpu.VMEM((_CHUNK // _SUB, _SUB, 128), jnp.float32),
          pltpu.SemaphoreType.DMA,
      ],
  )
  def k(a_hbm, m_hbm, g_hbm, idx_v, t_v, rows_v, pack_v, sem):
    wid = lax.axis_index("s") * 2 + lax.axis_index("c")
    base = wid * _BPW
    pltpu.sync_copy(a_hbm.at[pl.ds(base, _BPW)], idx_v)

    def shift_body(c, _):
      t_v[pl.ds(c * _L, _L)] = jnp.right_shift(idx_v[pl.ds(c * _L, _L)], 3)
      return _

    lax.fori_loop(0, _BPW // _L, shift_body, 0)

    for ch in range(_NCHUNK):
      c0 = ch * _CHUNK
      pltpu.async_copy(m_hbm.at[t_v.at[pl.ds(c0, _CHUNK)]], rows_v, sem).wait()

      def body(c, _):
        offs = jnp.bitwise_and(idx_v[pl.ds(c0 + c * _L, _L)], 7) * D
        for j in range(_L):
          i = c * _L + j
          pack_v[i // _SUB, j % _SUB, pl.ds(0, D)] = (
              rows_v[i, pl.ds(offs[j], D)])
        return _

      lax.fori_loop(0, _CHUNK // _L, body, 0)
      pltpu.sync_copy(
          pack_v,
          g_hbm.at[pl.ds(wid * (_BPW // _SUB) + ch * (_CHUNK // _SUB),
                         _CHUNK // _SUB)])

  return k


_gather = _make_gather_kernel()


def _dot_sigmoid_body(x_ref, g_ref, o_ref):
  g = g_ref[...][:, :, :D].reshape(_TC_ROWS, D)
  p = x_ref[...] * g
  z = jnp.sum(p, axis=1)
  o_ref[...] = 1.0 / (1.0 + jnp.exp(-z))


_dot_sigmoid = pl.pallas_call(
    _dot_sigmoid_body,
    grid=(B // _TC_ROWS,),
    in_specs=[
        pl.BlockSpec((_TC_ROWS, D), lambda i: (i, 0)),
        pl.BlockSpec((_TC_G, _SUB, 128), lambda i: (i, 0, 0)),
    ],
    out_specs=pl.BlockSpec((_TC_ROWS,), lambda i: (i,)),
    out_shape=jax.ShapeDtypeStruct((B,), jnp.float32),
)


@jax.jit
def kernel(X, A, m):
  eye = jnp.eye(D, dtype=jnp.float32)
  mt_p = jnp.pad(m.T, ((0, 0), (0, _KPAD - K)))
  m4 = _repack_m(mt_p, eye)
  g3 = _gather(A.astype(jnp.int32), m4)
  return _dot_sigmoid(X, g3)


# speedup vs baseline: 1.3118x; 1.1478x over previous
import functools

import jax
import jax.numpy as jnp
from jax import lax
from jax.experimental import pallas as pl
from jax.experimental.pallas import tpu as pltpu
from jax.experimental.pallas import tpu_sc as plsc

K = 100000
D = 16
B = 16384

_NW = 32
_BPW = B // _NW
_SUB = 8
_KT = K // _SUB
_G1 = B // _SUB
_L = 16
_NCHUNK = 2
_CHUNK = _BPW // _NCHUNK

_TC_ROWS = 2048
_TC_G = _TC_ROWS // _SUB


def _make_gather_kernel():
  mesh = plsc.VectorSubcoreMesh(core_axis_name="c", subcore_axis_name="s")

  @functools.partial(
      pl.kernel,
      mesh=mesh,
      compiler_params=pltpu.CompilerParams(use_tc_tiling_on_sc=False),
      out_type=jax.ShapeDtypeStruct((_G1, _SUB, 128), jnp.float32),
      scratch_types=[
          pltpu.VMEM((_BPW,), jnp.int32),
          pltpu.VMEM((_BPW, D), jnp.float32),
          pltpu.VMEM((_BPW // _SUB, _SUB, 128), jnp.float32),
          pltpu.SemaphoreType.DMA,
      ],
  )
  def k(a_hbm, m_hbm, g_hbm, idx_v, rows_v, pack_v, sem):
    wid = lax.axis_index("s") * 2 + lax.axis_index("c")
    base = wid * _BPW
    pltpu.sync_copy(a_hbm.at[pl.ds(base, _BPW)], idx_v)
    pltpu.async_copy(m_hbm.at[idx_v], rows_v, sem).wait()

    def body(t, _):
      for j in range(_SUB):
        pack_v[t, j, pl.ds(0, D)] = rows_v[t * _SUB + j, :]
      return _

    lax.fori_loop(0, _BPW // _SUB, body, 0)
    pltpu.sync_copy(pack_v, g_hbm.at[pl.ds(wid * (_BPW // _SUB), _BPW // _SUB)])

  return k


_gather = _make_gather_kernel()


def _dot_sigmoid_body(x_ref, g_ref, o_ref):
  g = g_ref[...][:, :, :D].reshape(_TC_ROWS, D)
  p = x_ref[...] * g
  z = jnp.sum(p, axis=1)
  o_ref[...] = 1.0 / (1.0 + jnp.exp(-z))


_dot_sigmoid = pl.pallas_call(
    _dot_sigmoid_body,
    grid=(B // _TC_ROWS,),
    in_specs=[
        pl.BlockSpec((_TC_ROWS, D), lambda i: (i, 0)),
        pl.BlockSpec((_TC_G, _SUB, 128), lambda i: (i, 0, 0)),
    ],
    out_specs=pl.BlockSpec((_TC_ROWS,), lambda i: (i,)),
    out_shape=jax.ShapeDtypeStruct((B,), jnp.float32),
)


@jax.jit
def kernel(X, A, m):
  g3 = _gather(A.astype(jnp.int32), m)
  return _dot_sigmoid(X, g3)
